# Initial kernel scaffold; baseline (speedup 1.0000x reference)
import jax
import jax.numpy as jnp
from jax.experimental import pallas as pl


def kernel(pred_map0, pred_map1, pred_map2):
    raise NotImplementedError("write your pallas kernel here")


# speedup vs baseline: 6.0841x; 6.0841x over previous
import numpy as np
import jax
import jax.numpy as jnp
from jax import lax
from jax.experimental import pallas as pl

_NUM_CLASSES = 80
_STRIDES = (32, 16, 8)
_FEAT_SIZES = ((19, 19), (38, 38), (76, 76))
_BASE_ANCHORS = (
    ((116, 90), (156, 198), (373, 326)),
    ((30, 61), (62, 45), (59, 119)),
    ((10, 13), (16, 30), (33, 23)),
)
_NMS_PRE = 1000
_CONF_THR = 0.005
_SCORE_THR = 0.05
_IOU_THR = 0.45
_MAX_PER_IMG = 100
_PRE_TOPK = 2000
_CLASS_OFFSET = 4096.0

_LEVEL_SIZES = tuple(h * w * 3 for (h, w) in _FEAT_SIZES)
_N_TOTAL = sum(_LEVEL_SIZES)
_N_PAD = ((_N_TOTAL + 127) // 128) * 128
_NMS_PAD = 2048


def _build_anchor_consts():
    cols = []
    strides = []
    for lvl, (h, w) in enumerate(_FEAT_SIZES):
        stride = _STRIDES[lvl]
        ys, xs = np.meshgrid(np.arange(h), np.arange(w), indexing='ij')
        cx = (xs.reshape(-1, 1) * stride).astype(np.float32)
        cy = (ys.reshape(-1, 1) * stride).astype(np.float32)
        per_anchor = []
        for (aw, ah) in _BASE_ANCHORS[lvl]:
            per_anchor.append(
                np.concatenate([cx - aw / 2.0, cy - ah / 2.0,
                                cx + aw / 2.0, cy + ah / 2.0], axis=1))
        a = np.stack(per_anchor, axis=1).reshape(-1, 4)
        cols.append(a.T)
        strides.append(np.full((1, a.shape[0]), stride, np.float32))
    anchors = np.concatenate(cols, axis=1)
    stride_row = np.concatenate(strides, axis=1)
    pad = _N_PAD - anchors.shape[1]
    anchors = np.pad(anchors, ((0, 0), (0, pad)))
    stride_row = np.pad(stride_row, ((0, 0), (0, pad)), constant_values=1.0)
    return jnp.asarray(anchors), jnp.asarray(stride_row)


_ANCHORS, _STRIDE_ROW = _build_anchor_consts()


def _decode_kernel(x_ref, a_ref, s_ref, box_ref, conf_ref, cls_ref):
    x = x_ref[...]
    a = a_ref[...]
    s = s_ref[...]
    ax = (a[0:1, :] + a[2:3, :]) * 0.5
    ay = (a[1:2, :] + a[3:4, :]) * 0.5
    aw = a[2:3, :] - a[0:1, :]
    ah = a[3:4, :] - a[1:2, :]
    xy0 = jax.nn.sigmoid(jax.nn.sigmoid(x[0:1, :]))
    xy1 = jax.nn.sigmoid(jax.nn.sigmoid(x[1:2, :]))
    px = (xy0 - 0.5) * s + ax
    py = (xy1 - 0.5) * s + ay
    pw = jnp.exp(x[2:3, :]) * aw
    ph = jnp.exp(x[3:4, :]) * ah
    box_ref[...] = jnp.concatenate(
        [px - pw * 0.5, py - ph * 0.5, px + pw * 0.5, py + ph * 0.5], axis=0)
    conf_ref[...] = jax.nn.sigmoid(x[4:5, :])
    cls_ref[...] = jax.nn.sigmoid(x[5:85, :])


def _nms_kernel(boxes_ref, clsf_ref, score_ref, out_ref):
    boxes = boxes_ref[...]
    clsf = clsf_ref[...]
    off = clsf * _CLASS_OFFSET
    x1 = boxes[0:1, :] + off
    y1 = boxes[1:2, :] + off
    x2 = boxes[2:3, :] + off
    y2 = boxes[3:4, :] + off
    areas = jnp.clip(x2 - x1, 0.0) * jnp.clip(y2 - y1, 0.0)
    col = lax.broadcasted_iota(jnp.int32, (1, _NMS_PAD), 1)
    ocol = lax.broadcasted_iota(jnp.int32, (1, 128), 1)

    def body(i, sc):
        m = jnp.max(sc)
        j = jnp.min(jnp.where(sc == m, col, _NMS_PAD))
        pick = col == j
        bx1 = jnp.sum(jnp.where(pick, x1, 0.0))
        by1 = jnp.sum(jnp.where(pick, y1, 0.0))
        bx2 = jnp.sum(jnp.where(pick, x2, 0.0))
        by2 = jnp.sum(jnp.where(pick, y2, 0.0))
        cj = jnp.sum(jnp.where(pick, clsf, 0.0))
        offj = cj * _CLASS_OFFSET
        ix1 = jnp.maximum(bx1, x1)
        iy1 = jnp.maximum(by1, y1)
        ix2 = jnp.minimum(bx2, x2)
        iy2 = jnp.minimum(by2, y2)
        inter = jnp.clip(ix2 - ix1, 0.0) * jnp.clip(iy2 - iy1, 0.0)
        barea = jnp.clip(bx2 - bx1, 0.0) * jnp.clip(by2 - by1, 0.0)
        iou = inter / (barea + areas - inter + 1e-6)
        sc = jnp.where(iou > _IOU_THR, -1.0, sc)
        sc = jnp.where(pick, -1.0, sc)
        row = jnp.where(
            ocol == 0, m,
            jnp.where(ocol == 1, cj,
                      jnp.where(ocol == 2, bx1 - offj,
                                jnp.where(ocol == 3, by1 - offj,
                                          jnp.where(ocol == 4, bx2 - offj,
                                                    jnp.where(ocol == 5,
                                                              by2 - offj,
                                                              0.0))))))
        out_ref[pl.ds(i, 1), :] = row
        return sc

    lax.fori_loop(0, _MAX_PER_IMG, body, score_ref[...])


def _prep_level(pm, h, w):
    arr = pm.reshape(3, 85, h * w)
    return arr.transpose(1, 2, 0).reshape(85, h * w * 3)


def kernel(pred_map0, pred_map1, pred_map2):
    maps = (pred_map0, pred_map1, pred_map2)
    x = jnp.concatenate(
        [_prep_level(maps[i][0], *_FEAT_SIZES[i]) for i in range(3)], axis=1)
    x = jnp.pad(x, ((0, 0), (0, _N_PAD - _N_TOTAL)))

    boxes, conf, cls = pl.pallas_call(
        _decode_kernel,
        out_shape=[
            jax.ShapeDtypeStruct((4, _N_PAD), jnp.float32),
            jax.ShapeDtypeStruct((1, _N_PAD), jnp.float32),
            jax.ShapeDtypeStruct((_NUM_CLASSES, _N_PAD), jnp.float32),
        ],
    )(x, _ANCHORS, _STRIDE_ROW)

    conf = conf[0]
    inds = []
    confs = []
    off = 0
    for n in _LEVEL_SIZES:
        ct, ci = lax.top_k(conf[off:off + n], _NMS_PRE)
        inds.append(ci + off)
        confs.append(ct * (ct >= _CONF_THR).astype(ct.dtype))
        off += n
    inds = jnp.concatenate(inds)
    cand_conf = jnp.concatenate(confs)
    cand_boxes = boxes[:, inds]
    cand_cls = cls[:, inds]
    scores = cand_cls * cand_conf[None, :]
    scores = scores * (scores > _SCORE_THR).astype(scores.dtype)
    flat = scores.T.reshape(-1)
    top_scores, top_idx = lax.top_k(flat, _PRE_TOPK)
    box_idx = top_idx // _NUM_CLASSES
    cls_idx = top_idx % _NUM_CLASSES

    nb = jnp.pad(cand_boxes[:, box_idx], ((0, 0), (0, _NMS_PAD - _PRE_TOPK)))
    ncls = jnp.pad(cls_idx.astype(jnp.float32), (0, _NMS_PAD - _PRE_TOPK))[None, :]
    nsc = jnp.pad(top_scores, (0, _NMS_PAD - _PRE_TOPK),
                  constant_values=-1e30)[None, :]

    out = pl.pallas_call(
        _nms_kernel,
        out_shape=jax.ShapeDtypeStruct((_MAX_PER_IMG + 28, 128), jnp.float32),
    )(nb, ncls, nsc)

    sel_s = out[:_MAX_PER_IMG, 0]
    sel_cls = out[:_MAX_PER_IMG, 1]
    sel_box = out[:_MAX_PER_IMG, 2:6]
    validb = sel_s > _SCORE_THR
    valid = validb.astype(jnp.float32)
    proposals = sel_box * valid[:, None]
    out_scores = sel_s * valid
    out_cls = jnp.where(validb, sel_cls.astype(jnp.int32), -1)
    num_det = jnp.sum(valid).astype(jnp.int32)
    return (num_det.reshape(1), proposals[None], out_scores[None],
            out_cls[None])
